# Initial kernel scaffold; baseline (speedup 1.0000x reference)
import jax
import jax.numpy as jnp
from jax.experimental import pallas as pl


def kernel(Q, K, V):
    raise NotImplementedError("write your pallas kernel here")


# speedup vs baseline: 1.0000x; 1.0000x over previous
import math
import jax
import jax.numpy as jnp
from jax.experimental import pallas as pl


def kernel(Q, K, V):
    dk = dv = Q.shape[2]
    c = 2
    Bn, Ln, _ = Q.shape
    Q_freq = jnp.fft.rfft(Q, axis=1)
    K_freq = jnp.fft.rfft(K, axis=1)
    Rxx = jnp.fft.irfft(Q_freq * jnp.conjugate(K_freq), n=Ln, axis=1)
    k = min(int(math.floor(c * math.log(Ln))), Ln)
    Wk, Ik = jax.lax.top_k(jnp.moveaxis(Rxx, 1, -1), k)
    Wk = jax.nn.softmax(Wk, axis=-1)

    def _per_d(_w, _i, _v):
        return _w * jnp.roll(_v, -_i)
    _per_d_v = jax.vmap(_per_d, in_axes=(0, 0, 1), out_axes=1)

    def _per_B(_wk, _ik, _V):
        return _per_d_v(_wk, _ik, _V)
    _per_B_v = jax.vmap(_per_B)

    def _per_k(wk, ik):
        return _per_B_v(wk, ik, V)
    _per_k_v = jax.vmap(_per_k, in_axes=-1)

    A = jnp.sum(_per_k_v(Wk, Ik), axis=0)
    return A


# speedup vs baseline: 65.9415x; 65.9409x over previous
import functools
import math

import numpy as np
import jax
import jax.numpy as jnp
from jax import lax
from jax.experimental import pallas as pl
from jax.experimental.pallas import tpu as pltpu
from jax.experimental.pallas import tpu_sc as plsc

L = 4096
D = 768
B = 2
NF = 2304
KTOP = 16
NCH = B * D
NW = 32
CHW = NCH // NW

_DN = (((0,), (0,)), ((), ()))
_PREC = lax.Precision.HIGHEST


def _dft_tables():
    n = np.arange(L, dtype=np.int64)
    f = np.arange(NF, dtype=np.int64)
    ang = 2.0 * np.pi * ((n[:, None] * f[None, :]) % L).astype(np.float64) / L
    valid = (f < L // 2 + 1)[None, :]
    cct = np.where(valid, np.cos(ang), 0.0).astype(np.float32)
    cst = np.where(valid, np.sin(ang), 0.0).astype(np.float32)
    ang2 = 2.0 * np.pi * ((f[:, None] * n[None, :]) % L).astype(np.float64) / L
    alpha = np.where((f == 0) | (f == L // 2), 1.0, 2.0) / L
    alpha = np.where(f < L // 2 + 1, alpha, 0.0)[:, None]
    ic = (alpha * np.cos(ang2)).astype(np.float32)
    isn = (-alpha * np.sin(ang2)).astype(np.float32)
    return cct, cst, ic, isn


_CCT, _CST, _IC, _IS = _dft_tables()

DB = 128
FB = 256
LB = 256
CB = 256


def _corr_fwd(q_ref, k_ref, cc_ref, cs_ref, pr_ref, pi_ref):
    q = q_ref[0]
    k = k_ref[0]
    cc = cc_ref[...]
    cs = cs_ref[...]
    qc = lax.dot_general(q, cc, _DN, precision=_PREC,
                         preferred_element_type=jnp.float32)
    qs = lax.dot_general(q, cs, _DN, precision=_PREC,
                         preferred_element_type=jnp.float32)
    kc = lax.dot_general(k, cc, _DN, precision=_PREC,
                         preferred_element_type=jnp.float32)
    ks = lax.dot_general(k, cs, _DN, precision=_PREC,
                         preferred_element_type=jnp.float32)
    pr_ref[0] = qc * kc + qs * ks
    pi_ref[0] = qc * ks - qs * kc


def _corr_inv(pr_ref, pi_ref, ic_ref, is_ref, rxx_ref):
    pr = pr_ref[0]
    pi = pi_ref[0]
    dn = (((1,), (0,)), ((), ()))
    rxx = lax.dot_general(pr, ic_ref[...], dn, precision=_PREC,
                          preferred_element_type=jnp.float32)
    rxx += lax.dot_general(pi, is_ref[...], dn, precision=_PREC,
                           preferred_element_type=jnp.float32)
    rxx_ref[0] = rxx


def _topk(rxx_ref, w_ref, i_ref):
    x = rxx_ref[0]
    cols = lax.broadcasted_iota(jnp.int32, (CB, L), 1)
    neg = jnp.float32(-3.0e38)
    vals, idxs = [], []
    for _ in range(KTOP):
        m = jnp.max(x, axis=1, keepdims=True)
        idx = jnp.min(jnp.where(x == m, cols, L), axis=1, keepdims=True)
        vals.append(m)
        idxs.append(idx)
        x = jnp.where(cols == idx, neg, x)
    v = jnp.concatenate(vals, axis=1)
    ii = jnp.concatenate(idxs, axis=1)
    e = jnp.exp(v - v[:, 0:1])
    w_ref[0] = e / jnp.sum(e, axis=1, keepdims=True)
    i_ref[0] = ii


def _roll_sum(vt_hbm, w_hbm, i_hbm, out_hbm, vext, acc, wrow, irow):
    cid = lax.axis_index("c")
    sid = lax.axis_index("s")
    wid = sid * 2 + cid
    base = wid * CHW

    def per_channel(ci, carry):
        ch = base + ci
        pltpu.sync_copy(vt_hbm.at[ch], vext.at[pl.ds(0, L)])
        pltpu.sync_copy(vt_hbm.at[ch], vext.at[pl.ds(L, L)])
        pltpu.sync_copy(w_hbm.at[ch], wrow)
        pltpu.sync_copy(i_hbm.at[ch], irow)
        wv = wrow[...]
        iv = irow[...]
        ws = [wv[j] for j in range(KTOP)]
        sh = [iv[j] for j in range(KTOP)]

        def per_t(t, c2):
            toff = t * 16
            accv = jnp.zeros((16,), jnp.float32)
            for j in range(KTOP):
                accv = accv + ws[j] * vext[pl.ds(sh[j] + toff, 16)]
            acc[pl.ds(toff, 16)] = accv
            return c2

        lax.fori_loop(0, L // 16, per_t, 0)
        pltpu.sync_copy(acc, out_hbm.at[ch])
        return carry

    lax.fori_loop(0, CHW, per_channel, 0)


def kernel(Q, K, V):
    cct = jnp.asarray(_CCT)
    cst = jnp.asarray(_CST)
    ic = jnp.asarray(_IC)
    isn = jnp.asarray(_IS)

    pr, pi = pl.pallas_call(
        _corr_fwd,
        grid=(B, D // DB, NF // FB),
        in_specs=[
            pl.BlockSpec((1, L, DB), lambda b, c, f: (b, 0, c)),
            pl.BlockSpec((1, L, DB), lambda b, c, f: (b, 0, c)),
            pl.BlockSpec((L, FB), lambda b, c, f: (0, f)),
            pl.BlockSpec((L, FB), lambda b, c, f: (0, f)),
        ],
        out_specs=[
            pl.BlockSpec((1, DB, FB), lambda b, c, f: (b, c, f)),
            pl.BlockSpec((1, DB, FB), lambda b, c, f: (b, c, f)),
        ],
        out_shape=[
            jax.ShapeDtypeStruct((B, D, NF), jnp.float32),
            jax.ShapeDtypeStruct((B, D, NF), jnp.float32),
        ],
    )(Q, K, cct, cst)

    rxx = pl.pallas_call(
        _corr_inv,
        grid=(B, L // LB),
        in_specs=[
            pl.BlockSpec((1, D, NF), lambda b, l: (b, 0, 0)),
            pl.BlockSpec((1, D, NF), lambda b, l: (b, 0, 0)),
            pl.BlockSpec((NF, LB), lambda b, l: (0, l)),
            pl.BlockSpec((NF, LB), lambda b, l: (0, l)),
        ],
        out_specs=pl.BlockSpec((1, D, LB), lambda b, l: (b, 0, l)),
        out_shape=jax.ShapeDtypeStruct((B, D, L), jnp.float32),
    )(pr, pi, ic, isn)

    w, ik = pl.pallas_call(
        _topk,
        grid=(B, D // CB),
        in_specs=[pl.BlockSpec((1, CB, L), lambda b, c: (b, c, 0))],
        out_specs=[
            pl.BlockSpec((1, CB, KTOP), lambda b, c: (b, c, 0)),
            pl.BlockSpec((1, CB, KTOP), lambda b, c: (b, c, 0)),
        ],
        out_shape=[
            jax.ShapeDtypeStruct((B, D, KTOP), jnp.float32),
            jax.ShapeDtypeStruct((B, D, KTOP), jnp.int32),
        ],
    )(rxx)

    vt = jnp.swapaxes(V, 1, 2).reshape(NCH, L)
    wf = w.reshape(NCH, KTOP)
    inf = ik.reshape(NCH, KTOP)

    mesh = plsc.VectorSubcoreMesh(core_axis_name="c", subcore_axis_name="s")
    at = pl.kernel(
        _roll_sum,
        out_type=jax.ShapeDtypeStruct((NCH, L), jnp.float32),
        mesh=mesh,
        scratch_types=[
            pltpu.VMEM((2 * L,), jnp.float32),
            pltpu.VMEM((L,), jnp.float32),
            pltpu.VMEM((KTOP,), jnp.float32),
            pltpu.VMEM((KTOP,), jnp.int32),
        ],
    )(vt, wf, inf)

    return jnp.swapaxes(at.reshape(B, D, L), 1, 2)


# speedup vs baseline: 101.4719x; 1.5388x over previous
import functools
import math

import numpy as np
import jax
import jax.numpy as jnp
from jax import lax
from jax.experimental import pallas as pl
from jax.experimental.pallas import tpu as pltpu
from jax.experimental.pallas import tpu_sc as plsc

L = 4096
D = 768
B = 2
NF = 2304
KTOP = 16
NCH = B * D
NW = 32
CHW = NCH // NW

_DN = (((0,), (0,)), ((), ()))


def _split_bf16(x):
    hi = x.astype(jnp.bfloat16)
    lo = (x - hi.astype(jnp.float32)).astype(jnp.bfloat16)
    return hi, lo


def _dot3(xh, xl, mh, ml, dn):
    acc = lax.dot_general(xh, mh, dn, preferred_element_type=jnp.float32)
    acc += lax.dot_general(xh, ml, dn, preferred_element_type=jnp.float32)
    acc += lax.dot_general(xl, mh, dn, preferred_element_type=jnp.float32)
    return acc


def _dft_tables():
    n = np.arange(L, dtype=np.int64)
    f = np.arange(NF, dtype=np.int64)
    ang = 2.0 * np.pi * ((n[:, None] * f[None, :]) % L).astype(np.float64) / L
    valid = (f < L // 2 + 1)[None, :]
    cct = np.where(valid, np.cos(ang), 0.0).astype(np.float32)
    cst = np.where(valid, np.sin(ang), 0.0).astype(np.float32)
    ang2 = 2.0 * np.pi * ((f[:, None] * n[None, :]) % L).astype(np.float64) / L
    alpha = np.where((f == 0) | (f == L // 2), 1.0, 2.0) / L
    alpha = np.where(f < L // 2 + 1, alpha, 0.0)[:, None]
    ic = (alpha * np.cos(ang2)).astype(np.float32)
    isn = (-alpha * np.sin(ang2)).astype(np.float32)
    return cct, cst, ic, isn


def _np_split(x):
    import ml_dtypes
    hi = x.astype(ml_dtypes.bfloat16)
    lo = (x - hi.astype(np.float32)).astype(ml_dtypes.bfloat16)
    return hi, lo


_CCT, _CST, _IC, _IS = _dft_tables()
_CCH, _CCL = _np_split(_CCT)
_CSH, _CSL = _np_split(_CST)
_ICH, _ICL = _np_split(_IC)
_ISH, _ISL = _np_split(_IS)

DB = 128
FB = 256
LB = 256
CB = 256


def _corr_fwd(q_ref, k_ref, cch_ref, ccl_ref, csh_ref, csl_ref, pr_ref, pi_ref):
    qh, ql = _split_bf16(q_ref[0])
    kh, kl = _split_bf16(k_ref[0])
    cch, ccl = cch_ref[...], ccl_ref[...]
    csh, csl = csh_ref[...], csl_ref[...]
    qc = _dot3(qh, ql, cch, ccl, _DN)
    qs = _dot3(qh, ql, csh, csl, _DN)
    kc = _dot3(kh, kl, cch, ccl, _DN)
    ks = _dot3(kh, kl, csh, csl, _DN)
    pr_ref[0] = qc * kc + qs * ks
    pi_ref[0] = qc * ks - qs * kc


def _corr_inv(pr_ref, pi_ref, ich_ref, icl_ref, ish_ref, isl_ref, rxx_ref):
    prh, prl = _split_bf16(pr_ref[0])
    pih, pil = _split_bf16(pi_ref[0])
    dn = (((1,), (0,)), ((), ()))
    rxx = _dot3(prh, prl, ich_ref[...], icl_ref[...], dn)
    rxx += _dot3(pih, pil, ish_ref[...], isl_ref[...], dn)
    rxx_ref[0] = rxx


def _topk(rxx_ref, w_ref, i_ref):
    x = rxx_ref[0]
    cols = lax.broadcasted_iota(jnp.int32, (CB, L), 1)
    neg = jnp.float32(-3.0e38)
    vals, idxs = [], []
    for _ in range(KTOP):
        m = jnp.max(x, axis=1, keepdims=True)
        idx = jnp.min(jnp.where(x == m, cols, L), axis=1, keepdims=True)
        vals.append(m)
        idxs.append(idx)
        x = jnp.where(cols == idx, neg, x)
    v = jnp.concatenate(vals, axis=1)
    ii = jnp.concatenate(idxs, axis=1)
    e = jnp.exp(v - v[:, 0:1])
    w_ref[0] = e / jnp.sum(e, axis=1, keepdims=True)
    i_ref[0] = ii


def _roll_sum(vt_hbm, w_hbm, i_hbm, out_hbm, vext, acc, wrow, irow):
    cid = lax.axis_index("c")
    sid = lax.axis_index("s")
    wid = sid * 2 + cid
    base = wid * CHW

    def per_channel(ci, carry):
        ch = base + ci
        pltpu.sync_copy(vt_hbm.at[ch], vext.at[pl.ds(0, L)])
        pltpu.sync_copy(vt_hbm.at[ch], vext.at[pl.ds(L, L)])
        pltpu.sync_copy(w_hbm.at[ch], wrow)
        pltpu.sync_copy(i_hbm.at[ch], irow)
        wv = wrow[...]
        iv = irow[...]
        ws = [wv[j] for j in range(KTOP)]
        sh = [iv[j] for j in range(KTOP)]

        def per_t(t, c2):
            toff = t * 16
            accv = jnp.zeros((16,), jnp.float32)
            for j in range(KTOP):
                accv = accv + ws[j] * vext[pl.ds(sh[j] + toff, 16)]
            acc[pl.ds(toff, 16)] = accv
            return c2

        lax.fori_loop(0, L // 16, per_t, 0)
        pltpu.sync_copy(acc, out_hbm.at[ch])
        return carry

    lax.fori_loop(0, CHW, per_channel, 0)


def kernel(Q, K, V):
    fwd_mats = [jnp.asarray(m) for m in (_CCH, _CCL, _CSH, _CSL)]
    inv_mats = [jnp.asarray(m) for m in (_ICH, _ICL, _ISH, _ISL)]

    pr, pi = pl.pallas_call(
        _corr_fwd,
        grid=(B, D // DB, NF // FB),
        in_specs=[
            pl.BlockSpec((1, L, DB), lambda b, c, f: (b, 0, c)),
            pl.BlockSpec((1, L, DB), lambda b, c, f: (b, 0, c)),
        ] + [pl.BlockSpec((L, FB), lambda b, c, f: (0, f))] * 4,
        out_specs=[
            pl.BlockSpec((1, DB, FB), lambda b, c, f: (b, c, f)),
            pl.BlockSpec((1, DB, FB), lambda b, c, f: (b, c, f)),
        ],
        out_shape=[
            jax.ShapeDtypeStruct((B, D, NF), jnp.float32),
            jax.ShapeDtypeStruct((B, D, NF), jnp.float32),
        ],
    )(Q, K, *fwd_mats)

    rxx = pl.pallas_call(
        _corr_inv,
        grid=(B, L // LB),
        in_specs=[
            pl.BlockSpec((1, D, NF), lambda b, l: (b, 0, 0)),
            pl.BlockSpec((1, D, NF), lambda b, l: (b, 0, 0)),
        ] + [pl.BlockSpec((NF, LB), lambda b, l: (0, l))] * 4,
        out_specs=pl.BlockSpec((1, D, LB), lambda b, l: (b, 0, l)),
        out_shape=jax.ShapeDtypeStruct((B, D, L), jnp.float32),
    )(pr, pi, *inv_mats)

    w, ik = pl.pallas_call(
        _topk,
        grid=(B, D // CB),
        in_specs=[pl.BlockSpec((1, CB, L), lambda b, c: (b, c, 0))],
        out_specs=[
            pl.BlockSpec((1, CB, KTOP), lambda b, c: (b, c, 0)),
            pl.BlockSpec((1, CB, KTOP), lambda b, c: (b, c, 0)),
        ],
        out_shape=[
            jax.ShapeDtypeStruct((B, D, KTOP), jnp.float32),
            jax.ShapeDtypeStruct((B, D, KTOP), jnp.int32),
        ],
    )(rxx)

    vt = jnp.swapaxes(V, 1, 2).reshape(NCH, L)
    wf = w.reshape(NCH, KTOP)
    inf = ik.reshape(NCH, KTOP)

    mesh = plsc.VectorSubcoreMesh(core_axis_name="c", subcore_axis_name="s")
    at = pl.kernel(
        _roll_sum,
        out_type=jax.ShapeDtypeStruct((NCH, L), jnp.float32),
        mesh=mesh,
        scratch_types=[
            pltpu.VMEM((2 * L,), jnp.float32),
            pltpu.VMEM((L,), jnp.float32),
            pltpu.VMEM((KTOP,), jnp.float32),
            pltpu.VMEM((KTOP,), jnp.int32),
        ],
    )(vt, wf, inf)

    return jnp.swapaxes(at.reshape(B, D, L), 1, 2)


# speedup vs baseline: 118.3632x; 1.1665x over previous
import functools
import math

import numpy as np
import jax
import jax.numpy as jnp
from jax import lax
from jax.experimental import pallas as pl
from jax.experimental.pallas import tpu as pltpu
from jax.experimental.pallas import tpu_sc as plsc

L = 4096
D = 768
B = 2
NF = 2304
KTOP = 16
NCH = B * D
NW = 32
CHW = NCH // NW

_DN = (((0,), (0,)), ((), ()))


def _split_bf16(x):
    hi = x.astype(jnp.bfloat16)
    lo = (x - hi.astype(jnp.float32)).astype(jnp.bfloat16)
    return hi, lo


def _dot3(xh, xl, mh, ml, dn):
    acc = lax.dot_general(xh, mh, dn, preferred_element_type=jnp.float32)
    acc += lax.dot_general(xh, ml, dn, preferred_element_type=jnp.float32)
    acc += lax.dot_general(xl, mh, dn, preferred_element_type=jnp.float32)
    return acc


def _dft_tables():
    n = np.arange(L, dtype=np.int64)
    f = np.arange(NF, dtype=np.int64)
    ang = 2.0 * np.pi * ((n[:, None] * f[None, :]) % L).astype(np.float64) / L
    valid = (f < L // 2 + 1)[None, :]
    cct = np.where(valid, np.cos(ang), 0.0).astype(np.float32)
    cst = np.where(valid, np.sin(ang), 0.0).astype(np.float32)
    ang2 = 2.0 * np.pi * ((f[:, None] * n[None, :]) % L).astype(np.float64) / L
    alpha = np.where((f == 0) | (f == L // 2), 1.0, 2.0) / L
    alpha = np.where(f < L // 2 + 1, alpha, 0.0)[:, None]
    ic = (alpha * np.cos(ang2)).astype(np.float32)
    isn = (-alpha * np.sin(ang2)).astype(np.float32)
    return cct, cst, ic, isn


def _np_split(x):
    import ml_dtypes
    hi = x.astype(ml_dtypes.bfloat16)
    lo = (x - hi.astype(np.float32)).astype(ml_dtypes.bfloat16)
    return hi, lo


_CCT, _CST, _IC, _IS = _dft_tables()
_CCH, _CCL = _np_split(_CCT)
_CSH, _CSL = _np_split(_CST)
_ICH, _ICL = _np_split(_IC)
_ISH, _ISL = _np_split(_IS)

DB = 128
FB = 256
LB = 256
CB = 256


def _corr_fwd(q_ref, k_ref, cch_ref, ccl_ref, csh_ref, csl_ref, pr_ref, pi_ref):
    qh, ql = _split_bf16(q_ref[0])
    kh, kl = _split_bf16(k_ref[0])
    cch, ccl = cch_ref[...], ccl_ref[...]
    csh, csl = csh_ref[...], csl_ref[...]
    qc = _dot3(qh, ql, cch, ccl, _DN)
    qs = _dot3(qh, ql, csh, csl, _DN)
    kc = _dot3(kh, kl, cch, ccl, _DN)
    ks = _dot3(kh, kl, csh, csl, _DN)
    pr_ref[0] = qc * kc + qs * ks
    pi_ref[0] = qc * ks - qs * kc


def _corr_inv(pr_ref, pi_ref, ich_ref, icl_ref, ish_ref, isl_ref, rxx_ref):
    prh, prl = _split_bf16(pr_ref[0])
    pih, pil = _split_bf16(pi_ref[0])
    dn = (((1,), (0,)), ((), ()))
    rxx = _dot3(prh, prl, ich_ref[...], icl_ref[...], dn)
    rxx += _dot3(pih, pil, ish_ref[...], isl_ref[...], dn)
    rxx_ref[0] = rxx


def _topk(rxx_ref, w_ref, i_ref):
    x = rxx_ref[0]
    cols = lax.broadcasted_iota(jnp.int32, (CB, L), 1)
    neg = jnp.float32(-3.0e38)
    vals, idxs = [], []
    for _ in range(KTOP):
        m = jnp.max(x, axis=1, keepdims=True)
        idx = jnp.min(jnp.where(x == m, cols, L), axis=1, keepdims=True)
        vals.append(m)
        idxs.append(idx)
        x = jnp.where(cols == idx, neg, x)
    v = jnp.concatenate(vals, axis=1)
    ii = jnp.concatenate(idxs, axis=1)
    e = jnp.exp(v - v[:, 0:1])
    w_ref[0] = e / jnp.sum(e, axis=1, keepdims=True)
    i_ref[0] = ii


def _roll_sum(vt_hbm, w_hbm, i_hbm, out_hbm, vext, acc, wrow, irow):
    cid = lax.axis_index("c")
    sid = lax.axis_index("s")
    wid = sid * 2 + cid
    base = wid * CHW

    wthresh = jnp.float32(1e-6)

    def per_channel(ci, carry):
        ch = base + ci
        pltpu.sync_copy(vt_hbm.at[ch], vext.at[pl.ds(0, L)])
        pltpu.sync_copy(vt_hbm.at[ch], vext.at[pl.ds(L, L)])
        pltpu.sync_copy(w_hbm.at[ch], wrow)
        pltpu.sync_copy(i_hbm.at[ch], irow)
        wv = wrow[...]
        iv = irow[...]
        ws = [wv[j] for j in range(KTOP)]
        sh = [iv[j] for j in range(KTOP)]

        @pl.loop(0, L // 16, unroll=4)
        def _t0(t):
            toff = t * 16
            accv = ws[0] * vext[pl.ds(sh[0] + toff, 16)]
            for j in range(1, 4):
                accv += ws[j] * vext[pl.ds(sh[j] + toff, 16)]
            acc[pl.ds(toff, 16)] = accv

        for q in range(1, KTOP // 4):
            @pl.when(ws[4 * q] >= wthresh)
            def _quad(q=q):
                @pl.loop(0, L // 16, unroll=4)
                def _t(t):
                    toff = t * 16
                    accv = acc[pl.ds(toff, 16)]
                    for j in range(4 * q, 4 * q + 4):
                        accv += ws[j] * vext[pl.ds(sh[j] + toff, 16)]
                    acc[pl.ds(toff, 16)] = accv

        pltpu.sync_copy(acc, out_hbm.at[ch])
        return carry

    lax.fori_loop(0, CHW, per_channel, 0)


def kernel(Q, K, V):
    fwd_mats = [jnp.asarray(m) for m in (_CCH, _CCL, _CSH, _CSL)]
    inv_mats = [jnp.asarray(m) for m in (_ICH, _ICL, _ISH, _ISL)]

    pr, pi = pl.pallas_call(
        _corr_fwd,
        grid=(B, D // DB, NF // FB),
        in_specs=[
            pl.BlockSpec((1, L, DB), lambda b, c, f: (b, 0, c)),
            pl.BlockSpec((1, L, DB), lambda b, c, f: (b, 0, c)),
        ] + [pl.BlockSpec((L, FB), lambda b, c, f: (0, f))] * 4,
        out_specs=[
            pl.BlockSpec((1, DB, FB), lambda b, c, f: (b, c, f)),
            pl.BlockSpec((1, DB, FB), lambda b, c, f: (b, c, f)),
        ],
        out_shape=[
            jax.ShapeDtypeStruct((B, D, NF), jnp.float32),
            jax.ShapeDtypeStruct((B, D, NF), jnp.float32),
        ],
    )(Q, K, *fwd_mats)

    rxx = pl.pallas_call(
        _corr_inv,
        grid=(B, L // LB),
        in_specs=[
            pl.BlockSpec((1, D, NF), lambda b, l: (b, 0, 0)),
            pl.BlockSpec((1, D, NF), lambda b, l: (b, 0, 0)),
        ] + [pl.BlockSpec((NF, LB), lambda b, l: (0, l))] * 4,
        out_specs=pl.BlockSpec((1, D, LB), lambda b, l: (b, 0, l)),
        out_shape=jax.ShapeDtypeStruct((B, D, L), jnp.float32),
    )(pr, pi, *inv_mats)

    w, ik = pl.pallas_call(
        _topk,
        grid=(B, D // CB),
        in_specs=[pl.BlockSpec((1, CB, L), lambda b, c: (b, c, 0))],
        out_specs=[
            pl.BlockSpec((1, CB, KTOP), lambda b, c: (b, c, 0)),
            pl.BlockSpec((1, CB, KTOP), lambda b, c: (b, c, 0)),
        ],
        out_shape=[
            jax.ShapeDtypeStruct((B, D, KTOP), jnp.float32),
            jax.ShapeDtypeStruct((B, D, KTOP), jnp.int32),
        ],
    )(rxx)

    vt = jnp.swapaxes(V, 1, 2).reshape(NCH, L)
    wf = w.reshape(NCH, KTOP)
    inf = ik.reshape(NCH, KTOP)

    mesh = plsc.VectorSubcoreMesh(core_axis_name="c", subcore_axis_name="s")
    at = pl.kernel(
        _roll_sum,
        out_type=jax.ShapeDtypeStruct((NCH, L), jnp.float32),
        mesh=mesh,
        scratch_types=[
            pltpu.VMEM((2 * L,), jnp.float32),
            pltpu.VMEM((L,), jnp.float32),
            pltpu.VMEM((KTOP,), jnp.float32),
            pltpu.VMEM((KTOP,), jnp.int32),
        ],
    )(vt, wf, inf)

    return jnp.swapaxes(at.reshape(B, D, L), 1, 2)
